# baseline (speedup 1.0000x reference)
import functools

import numpy as np
import jax
import jax.numpy as jnp
from jax import lax
from jax.experimental import pallas as pl
from jax.experimental.pallas import tpu as pltpu
from jax.experimental.pallas import tpu_sc as plsc


@functools.lru_cache(maxsize=None)
def _make_sc_gather(V2, D2, B):
    info = plsc.get_sparse_core_info()
    NC, NS = info.num_cores, info.num_subcores
    NW = NC * NS
    assert B % NW == 0
    b_per_w = B // NW
    CH = 128 if b_per_w % 128 == 0 else b_per_w
    n_ch = b_per_w // CH
    mesh = plsc.VectorSubcoreMesh(core_axis_name="c", subcore_axis_name="s")

    @functools.partial(
        pl.kernel,
        mesh=mesh,
        out_type=jax.ShapeDtypeStruct((B, D2), jnp.float32),
        scratch_types=[
            pltpu.VMEM((b_per_w,), jnp.int32),
            pltpu.VMEM((b_per_w, D2), jnp.float32),
            pltpu.SemaphoreType.DMA,
        ],
    )
    def gather_kernel(table_hbm, idx_hbm, out_hbm, idx_v, rows_v, sem):
        wid = lax.axis_index("s") * NC + lax.axis_index("c")
        base = wid * b_per_w
        pltpu.sync_copy(idx_hbm.at[pl.ds(base, b_per_w)], idx_v)
        copies = []
        for j in range(n_ch):
            c = pltpu.make_async_copy(
                table_hbm.at[idx_v.at[pl.ds(j * CH, CH)]],
                rows_v.at[pl.ds(j * CH, CH)],
                sem,
            )
            c.start()
            copies.append(c)
        for c in copies:
            c.wait()
        pltpu.sync_copy(rows_v, out_hbm.at[pl.ds(base, b_per_w)])

    return gather_kernel


@functools.lru_cache(maxsize=None)
def _make_tc_select_posenc_add(B, D):
    blk = min(B, 1024)
    assert B % blk == 0
    grid = B // blk

    def body(g_ref, xc_ref, o_ref):
        pid = pl.program_id(0)
        pos = (
            lax.broadcasted_iota(jnp.int32, (blk, D), 0) + pid * blk
        ).astype(jnp.float32)
        col = lax.broadcasted_iota(jnp.int32, (blk, D), 1)
        denom = jnp.power(
            jnp.float32(10000.0), col.astype(jnp.float32) * jnp.float32(2.0 / D)
        )
        ang = pos / denom
        pe = jnp.where(col % 2 == 0, jnp.sin(ang), jnp.cos(ang))
        g = g_ref[...]
        odd = (xc_ref[...] % 2) == 1
        row = jnp.where(odd, g[:, D:], g[:, :D])
        o_ref[...] = row + pe

    return pl.pallas_call(
        body,
        grid=(grid,),
        in_specs=[
            pl.BlockSpec((blk, 2 * D), lambda i: (i, 0)),
            pl.BlockSpec((blk, 1), lambda i: (i, 0)),
        ],
        out_specs=pl.BlockSpec((blk, D), lambda i: (i, 0)),
        out_shape=jax.ShapeDtypeStruct((B, D), jnp.float32),
    )


def kernel(x, table):
    B = x.shape[0]
    V, D = table.shape
    xi = x.astype(jnp.int32)
    table2 = table.reshape(V // 2, 2 * D)
    gathered = _make_sc_gather(V // 2, 2 * D, B)(table2, xi >> 1)
    return _make_tc_select_posenc_add(B, D)(gathered, xi.reshape(B, 1))


# speedup vs baseline: 1.0034x; 1.0034x over previous
import functools

import numpy as np
import jax
import jax.numpy as jnp
from jax import lax
from jax.experimental import pallas as pl
from jax.experimental.pallas import tpu as pltpu
from jax.experimental.pallas import tpu_sc as plsc


@functools.lru_cache(maxsize=None)
def _make_sc_gather(V2, D2, B):
    info = plsc.get_sparse_core_info()
    NC, NS = info.num_cores, info.num_subcores
    NW = NC * NS
    assert B % NW == 0
    b_per_w = B // NW
    CH = 128 if b_per_w % 128 == 0 else b_per_w
    n_ch = b_per_w // CH
    mesh = plsc.VectorSubcoreMesh(core_axis_name="c", subcore_axis_name="s")

    @functools.partial(
        pl.kernel,
        mesh=mesh,
        out_type=jax.ShapeDtypeStruct((B, D2), jnp.float32),
        scratch_types=[
            pltpu.VMEM((b_per_w,), jnp.int32),
            pltpu.VMEM((b_per_w, D2), jnp.float32),
            pltpu.SemaphoreType.DMA,
        ],
    )
    def gather_kernel(table_hbm, idx_hbm, out_hbm, idx_v, rows_v, sem):
        wid = lax.axis_index("s") * NC + lax.axis_index("c")
        base = wid * b_per_w
        pltpu.sync_copy(idx_hbm.at[pl.ds(base, b_per_w)], idx_v)

        @pl.loop(0, b_per_w // 16)
        def _half(i):
            v = idx_v[pl.ds(i * 16, 16)]
            idx_v[pl.ds(i * 16, 16)] = lax.shift_right_arithmetic(v, 1)

        copies = []
        for j in range(n_ch):
            c = pltpu.make_async_copy(
                table_hbm.at[idx_v.at[pl.ds(j * CH, CH)]],
                rows_v.at[pl.ds(j * CH, CH)],
                sem,
            )
            c.start()
            copies.append(c)
        for c in copies:
            c.wait()
        pltpu.sync_copy(rows_v, out_hbm.at[pl.ds(base, b_per_w)])

    return gather_kernel


@functools.lru_cache(maxsize=None)
def _make_tc_select_posenc_add(B, D):
    blk = min(B, 1024)
    assert B % blk == 0
    grid = B // blk

    def body(g_ref, xc_ref, o_ref):
        pid = pl.program_id(0)
        pos = (
            lax.broadcasted_iota(jnp.int32, (blk, D), 0) + pid * blk
        ).astype(jnp.float32)
        col = lax.broadcasted_iota(jnp.int32, (blk, D), 1)
        denom = jnp.power(
            jnp.float32(10000.0), col.astype(jnp.float32) * jnp.float32(2.0 / D)
        )
        ang = pos / denom
        pe = jnp.where(col % 2 == 0, jnp.sin(ang), jnp.cos(ang))
        g = g_ref[...]
        odd = (xc_ref[...] % 2) == 1
        row = jnp.where(odd, g[:, D:], g[:, :D])
        o_ref[...] = row + pe

    return pl.pallas_call(
        body,
        grid=(grid,),
        in_specs=[
            pl.BlockSpec((blk, 2 * D), lambda i: (i, 0)),
            pl.BlockSpec((blk, 1), lambda i: (i, 0)),
        ],
        out_specs=pl.BlockSpec((blk, D), lambda i: (i, 0)),
        out_shape=jax.ShapeDtypeStruct((B, D), jnp.float32),
    )


def kernel(x, table):
    B = x.shape[0]
    V, D = table.shape
    xi = x.astype(jnp.int32)
    table2 = table.reshape(V // 2, 2 * D)
    gathered = _make_sc_gather(V // 2, 2 * D, B)(table2, xi)
    return _make_tc_select_posenc_add(B, D)(gathered, xi.reshape(B, 1))


# speedup vs baseline: 1.0117x; 1.0082x over previous
import functools

import numpy as np
import jax
import jax.numpy as jnp
from jax import lax
from jax.experimental import pallas as pl
from jax.experimental.pallas import tpu as pltpu
from jax.experimental.pallas import tpu_sc as plsc


@functools.lru_cache(maxsize=None)
def _make_sc_gather(V2, D2, B):
    info = plsc.get_sparse_core_info()
    NC, NS = info.num_cores, info.num_subcores
    NW = NC * NS
    assert B % NW == 0
    b_per_w = B // NW
    CH = 128 if b_per_w % 128 == 0 else b_per_w
    n_ch = b_per_w // CH
    mesh = plsc.VectorSubcoreMesh(core_axis_name="c", subcore_axis_name="s")

    @functools.partial(
        pl.kernel,
        mesh=mesh,
        out_type=jax.ShapeDtypeStruct((B, D2), jnp.float32),
        scratch_types=[
            pltpu.VMEM((b_per_w,), jnp.int32),
            pltpu.VMEM((b_per_w, D2), jnp.float32),
            pltpu.SemaphoreType.DMA,
        ],
    )
    def gather_kernel(table_hbm, idx_hbm, out_hbm, idx_v, rows_v, sem):
        wid = lax.axis_index("s") * NC + lax.axis_index("c")
        base = wid * b_per_w
        pltpu.sync_copy(idx_hbm.at[pl.ds(base, b_per_w)], idx_v)

        @pl.loop(0, b_per_w // 16)
        def _half(i):
            v = idx_v[pl.ds(i * 16, 16)]
            idx_v[pl.ds(i * 16, 16)] = lax.shift_right_arithmetic(v, 1)

        copies = []
        for j in range(n_ch):
            c = pltpu.make_async_copy(
                table_hbm.at[idx_v.at[pl.ds(j * CH, CH)]],
                rows_v.at[pl.ds(j * CH, CH)],
                sem,
            )
            c.start()
            copies.append(c)
        for c in copies:
            c.wait()
        pltpu.sync_copy(rows_v, out_hbm.at[pl.ds(base, b_per_w)])

    return gather_kernel


@functools.lru_cache(maxsize=None)
def _make_tc_select_posenc_add(B, D):
    blk = min(B, 1024)
    assert B % blk == 0
    grid = B // blk

    def body(g_ref, xc_ref, o_ref):
        pid = pl.program_id(0)
        pos = (
            lax.broadcasted_iota(jnp.int32, (blk, D), 0) + pid * blk
        ).astype(jnp.float32)
        col = lax.broadcasted_iota(jnp.int32, (blk, D), 1)
        denom = jnp.power(
            jnp.float32(10000.0), col.astype(jnp.float32) * jnp.float32(2.0 / D)
        )
        ang = pos / denom
        pe = jnp.where(col % 2 == 0, jnp.sin(ang), jnp.cos(ang))
        g = g_ref[...]
        odd = (xc_ref[...] % 2) == 1
        row = jnp.where(odd, g[:, D:], g[:, :D])
        o_ref[...] = row + pe

    return pl.pallas_call(
        body,
        grid=(grid,),
        in_specs=[
            pl.BlockSpec((blk, 2 * D), lambda i: (i, 0)),
            pl.BlockSpec((blk, 1), lambda i: (i, 0)),
        ],
        out_specs=pl.BlockSpec((blk, D), lambda i: (i, 0)),
        out_shape=jax.ShapeDtypeStruct((B, D), jnp.float32),
    )


def kernel(x, table):
    B = x.shape[0]
    V, D = table.shape
    xi = x.astype(jnp.int32)
    table2 = table.reshape(V // 2, 2 * D)
    gathered = _make_sc_gather(V // 2, 2 * D, B)(table2, xi)
    pos = jnp.arange(B, dtype=jnp.float32)[:, None]
    col = jnp.arange(D)[None, :]
    ang = pos / jnp.power(10000.0, 2.0 * col.astype(jnp.float32) / D)
    pe = jnp.where(col % 2 == 0, jnp.sin(ang), jnp.cos(ang))
    odd = (xi[:, None] & 1) == 1
    row = jnp.where(odd, gathered[:, D:], gathered[:, :D])
    return row + pe
